# baseline (speedup 1.0000x reference)
import jax
import jax.numpy as jnp
from jax import lax
from jax.experimental import pallas as pl
from jax.experimental.pallas import tpu as pltpu
from jax.experimental.pallas import tpu_sc as plsc

B = 16384
NUM = 13
CAT = 26
TOK = NUM + CAT
V = 100000
D = 16

_info = plsc.get_sparse_core_info()
NC, NS = _info.num_cores, _info.num_subcores
NW = NC * NS
BPW = B // NW
G = 128
NGRP = BPW // G
H = 32
NH = G // H

NFT = V // 128
VTAIL = V - NFT * 128
RPF = 12504
VPAD = RPF * 128 // D
NTILE = CAT * NFT
NTI = (NTILE + NW - 1) // NW
NBT = B // 128
NBP = NBT // NW


def _fmt_body(tt_hbm, cat_hbm, num_hbm, tailq_hbm, tblf_hbm, idxt_hbm, xt_hbm,
              tb, vb, cbuf, nbuf, tvq, vbt, tsem, osem, csem):
    wid = lax.axis_index("s") * NC + lax.axis_index("c")
    lanes = jax.lax.iota(jnp.int32, 16)

    def _cmove(i, c):
        bt = wid + i * NW
        pltpu.sync_copy(cat_hbm.at[:, pl.ds(bt * 128, 128)],
                        cbuf.at[pl.ds(0, CAT)])
        pltpu.sync_copy(num_hbm.at[:, pl.ds(bt * 128, 128)],
                        nbuf.at[pl.ds(0, NUM)])
        pltpu.async_copy(cbuf, idxt_hbm.at[pl.ds(bt * 32, 32)], csem)
        pltpu.async_copy(nbuf, xt_hbm.at[pl.ds(bt * 16, 16)], csem)
        pltpu.make_async_copy(cbuf, idxt_hbm.at[pl.ds(bt * 32, 32)],
                              csem).wait()
        pltpu.make_async_copy(nbuf, xt_hbm.at[pl.ds(bt * 16, 16)],
                              csem).wait()
        return c
    lax.fori_loop(0, NBP, _cmove, 0)

    @pl.when(wid < CAT)
    def _():
        pltpu.sync_copy(tailq_hbm, tvq)
        for r in range(4):
            for kk in range(8):
                sl = pl.ds(kk * 16, 16)
                vbt[r, sl] = tvq[wid * 4 + r, sl]
        pltpu.sync_copy(vbt,
                        tblf_hbm.at[pl.ds(wid * RPF + NFT * 16, 8)])

    def _in_slices(p):
        f = p // NFT
        vt = p - f * NFT
        return (tt_hbm.at[f, :, pl.ds(pl.multiple_of(vt * 128, 128), 128)],
                tblf_hbm.at[pl.ds(f * RPF + vt * 16, 16)])

    def _fire_in(i, s):
        p = wid + i * NW

        @pl.when(p < NTILE)
        def _():
            src, _ = _in_slices(p)
            pltpu.async_copy(src, tb.at[s, :, pl.ds(0, 128)], tsem)

    _fire_in(0, 0)

    def _tile(i, c):
        s = i % 2
        p = wid + i * NW

        @pl.when(p < NTILE)
        def _():
            src, dst = _in_slices(p)
            pltpu.make_async_copy(src, tb.at[s, :, pl.ds(0, 128)], tsem).wait()
        _fire_in(i + 1, (i + 1) % 2)

        @pl.when(i >= 2)
        def _():
            pprev = wid + (i - 2) * NW
            _, dprev = _in_slices(pprev)
            pltpu.make_async_copy(vb.at[s], dprev, osem).wait()

        @pl.when(p < NTILE)
        def _():
            for bb in range(8):
                for t in range(16):
                    r = bb * 16 + t
                    vals = plsc.load_gather(
                        tb.at[s], [lanes, jnp.full((16,), r, jnp.int32)])
                    vb[s, r >> 3, pl.ds((r % 8) * 16, 16)] = vals
            _, dst = _in_slices(p)
            pltpu.async_copy(vb.at[s], dst, osem)
        return c

    lax.fori_loop(0, NTI, _tile, 0)

    for k in (2, 1):
        i = NTI - k

        @pl.when((i >= 0) & (wid + i * NW < NTILE))
        def _(i=i):
            p = wid + i * NW
            _, dst = _in_slices(p)
            pltpu.make_async_copy(vb.at[i % 2], dst, osem).wait()


@jax.jit
def _fmt(tt, catT, xT, tailq):
    mesh = plsc.VectorSubcoreMesh(core_axis_name="c", subcore_axis_name="s")
    return pl.kernel(
        _fmt_body,
        out_type=(
            jax.ShapeDtypeStruct((CAT * RPF, 128), jnp.float32),
            jax.ShapeDtypeStruct((NBT * 32, 128), jnp.int32),
            jax.ShapeDtypeStruct((NBT * 16, 128), jnp.float32),
        ),
        mesh=mesh,
        scratch_types=[
            pltpu.VMEM((2, D, 129), jnp.float32),
            pltpu.VMEM((2, D, 128), jnp.float32),
            pltpu.VMEM((32, 128), jnp.int32),
            pltpu.VMEM((16, 128), jnp.float32),
            pltpu.VMEM((CAT * 4, 128), jnp.float32),
            pltpu.VMEM((8, 128), jnp.float32),
            pltpu.SemaphoreType.DMA,
            pltpu.SemaphoreType.DMA,
            pltpu.SemaphoreType.DMA,
        ],
        compiler_params=pltpu.CompilerParams(use_tc_tiling_on_sc=True,
                                             needs_layout_passes=False),
    )(tt, catT, xT, tailq)


def _tok_body(xt_hbm, idxt_hbm, tbl_hbm, w_hbm, bias_hbm, out_hbm,
              idxt_v, xt_v, w_v, bias_v, gbuf, stage, gsem0, gsem1, osem):
    wid = lax.axis_index("s") * NC + lax.axis_index("c")
    base = wid * BPW

    pltpu.sync_copy(w_hbm, w_v)
    pltpu.sync_copy(bias_hbm, bias_v)

    lanes = jax.lax.iota(jnp.int32, 16)
    lanes3 = lanes >> 3
    icol = (lanes & 7) * 128
    gsems = (gsem0, gsem1)

    def _fire(h, hb):
        off = pl.multiple_of(h * H, H)
        for j in range(CAT):
            pltpu.async_copy(tbl_hbm.at[idxt_v.at[j, pl.ds(off, H)]],
                             gbuf.at[hb, j], gsems[hb])

    def _drain(h, hb):
        off = pl.multiple_of(h * H, H)
        for j in range(CAT):
            pltpu.make_async_copy(tbl_hbm.at[idxt_v.at[j, pl.ds(off, H)]],
                                  gbuf.at[hb, j], gsems[hb]).wait()

    def _group(g, carry):
        btv = wid * NGRP + g
        pltpu.sync_copy(idxt_hbm.at[btv, pl.ds(0, CAT)], idxt_v)
        pltpu.sync_copy(xt_hbm.at[btv, pl.ds(0, NUM)], xt_v)

        for j in range(CAT):
            for kk in range(G // 16):
                sl = pl.ds(kk * 16, 16)
                idxt_v[j, sl] = idxt_v[j, sl] + j * VPAD

        _fire(0, 0)

        @pl.when(g > 0)
        def _():
            pltpu.make_async_copy(
                stage, out_hbm.at[:, wid * NGRP + g - 1],
                osem).wait()

        for n in range(NUM):
            wv = w_v[n]
            bv = bias_v[n]
            for dt in range(2):
                for din in range(8):
                    d = dt * 8 + din
                    sw = wv[d]
                    sb = bv[d]

                    def _numk(k, c, n=n, dt=dt, din=din, sw=sw, sb=sb):
                        koff = pl.multiple_of(k * 16, 16)
                        xv = xt_v[n, pl.ds(koff, 16)]
                        stage[n * 2 + dt,
                              pl.ds(din * 128 + k * 16, 16)] = xv * sw + sb
                        return c
                    lax.fori_loop(0, G // 16, _numk, 0)

        for h in range(NH):
            hb = h % 2
            _drain(h, hb)
            if h + 1 < NH:
                _fire(h + 1, (h + 1) % 2)

            def _scat(bb, c, h=h, hb=hb):
                colv = icol + (h * H + bb)
                for j in range(CAT):
                    rowv = lanes3 + (NUM + j) * 2
                    plsc.store_scatter(stage, [rowv, colv], gbuf[hb, j, bb])
                return c
            lax.fori_loop(0, H, _scat, 0)

        pltpu.async_copy(stage, out_hbm.at[:, wid * NGRP + g],
                         osem)
        return carry

    lax.fori_loop(0, NGRP, _group, 0)
    pltpu.make_async_copy(
        stage, out_hbm.at[:, wid * NGRP + NGRP - 1], osem).wait()


@jax.jit
def _tok(xt, idxt, tbl, W, b):
    mesh = plsc.VectorSubcoreMesh(core_axis_name="c", subcore_axis_name="s")
    return pl.kernel(
        _tok_body,
        out_type=jax.ShapeDtypeStruct((TOK * 2, B // G, 8 * 128), jnp.float32),
        mesh=mesh,
        scratch_types=[
            pltpu.VMEM((CAT, G), jnp.int32),
            pltpu.VMEM((NUM, G), jnp.float32),
            pltpu.VMEM((NUM, D), jnp.float32),
            pltpu.VMEM((NUM, D), jnp.float32),
            pltpu.VMEM((2, CAT, H, D), jnp.float32),
            pltpu.VMEM((TOK * 2, 8 * 128), jnp.float32),
            pltpu.SemaphoreType.DMA,
            pltpu.SemaphoreType.DMA,
            pltpu.SemaphoreType.DMA,
        ],
        compiler_params=pltpu.CompilerParams(use_tc_tiling_on_sc=False,
                                             needs_layout_passes=False),
    )(xt, idxt, tbl, W, b)


def kernel(numerical_features, categorical_features, tables, W, b):
    tt = tables.transpose(0, 2, 1)
    catT = categorical_features.T
    xT = numerical_features.T
    tailq = lax.slice(tables, (0, V - VTAIL, 0), (CAT, V, D))
    tailq = tailq.reshape(CAT * VTAIL * D // 128, 128)
    tblf, idxt, xt = _fmt(tt, catT, xT, tailq)
    out = _tok(xt.reshape(B // G, 16, 128), idxt.reshape(B // G, 32, 128),
               tblf.reshape(CAT * VPAD, D), W, b)
    out = out.reshape(TOK, 2, B // G, 8, 128)
    out = out.transpose(2, 4, 0, 1, 3)
    return out.reshape(B, TOK, D)


# speedup vs baseline: 1.1376x; 1.1376x over previous
import jax
import jax.numpy as jnp
from jax import lax
from jax.experimental import pallas as pl
from jax.experimental.pallas import tpu as pltpu
from jax.experimental.pallas import tpu_sc as plsc

B = 16384
NUM = 13
CAT = 26
TOK = NUM + CAT
V = 100000
D = 16

_info = plsc.get_sparse_core_info()
NC, NS = _info.num_cores, _info.num_subcores
NW = NC * NS
BPW = B // NW
G = 128
NGRP = BPW // G
H = 32
NH = G // H

NFT = V // 128
VTAIL = V - NFT * 128
RPF = 12504
VPAD = RPF * 128 // D
NTILE = CAT * NFT
NTI = (NTILE + NW - 1) // NW
NBT = B // 128
NIB = 6
NOB = 3
NBP = NBT // NW


def _fmt_body(tt_hbm, cat_hbm, num_hbm, tailq_hbm, tblf_hbm, idxt_hbm, xt_hbm,
              tb, vb, cbuf, nbuf, tvq, vbt, tsem, osem, csem):
    wid = lax.axis_index("s") * NC + lax.axis_index("c")
    lanes = jax.lax.iota(jnp.int32, 16)

    def _cmove(i, c):
        bt = wid + i * NW
        pltpu.sync_copy(cat_hbm.at[:, pl.ds(bt * 128, 128)],
                        cbuf.at[pl.ds(0, CAT)])
        pltpu.sync_copy(num_hbm.at[:, pl.ds(bt * 128, 128)],
                        nbuf.at[pl.ds(0, NUM)])
        pltpu.async_copy(cbuf, idxt_hbm.at[pl.ds(bt * 32, 32)], csem)
        pltpu.async_copy(nbuf, xt_hbm.at[pl.ds(bt * 16, 16)], csem)
        pltpu.make_async_copy(cbuf, idxt_hbm.at[pl.ds(bt * 32, 32)],
                              csem).wait()
        pltpu.make_async_copy(nbuf, xt_hbm.at[pl.ds(bt * 16, 16)],
                              csem).wait()
        return c
    lax.fori_loop(0, NBP, _cmove, 0)

    @pl.when(wid < CAT)
    def _():
        pltpu.sync_copy(tailq_hbm, tvq)
        for r in range(4):
            for kk in range(8):
                sl = pl.ds(kk * 16, 16)
                vbt[r, sl] = tvq[wid * 4 + r, sl]
        pltpu.sync_copy(vbt,
                        tblf_hbm.at[pl.ds(wid * RPF + NFT * 16, 8)])

    def _in_slices(p):
        f = p // NFT
        vt = p - f * NFT
        return (tt_hbm.at[f, :, pl.ds(pl.multiple_of(vt * 128, 128), 128)],
                tblf_hbm.at[pl.ds(f * RPF + vt * 16, 16)])

    def _fire_in(i, s):
        p = wid + i * NW

        @pl.when(p < NTILE)
        def _():
            src, _ = _in_slices(p)
            pltpu.async_copy(src, tb.at[s], tsem)

    for q in range(NIB):
        _fire_in(q, q)

    def _tile(i, c):
        s = i % NIB
        p = wid + i * NW

        @pl.when(p < NTILE)
        def _():
            src, dst = _in_slices(p)
            pltpu.make_async_copy(src, tb.at[s], tsem).wait()
        _fire_in(i + NIB, (i + NIB) % NIB)

        so = i % NOB

        @pl.when(i >= NOB)
        def _():
            pprev = wid + (i - NOB) * NW
            _, dprev = _in_slices(pprev)
            pltpu.make_async_copy(vb.at[so], dprev, osem).wait()

        @pl.when(p < NTILE)
        def _():
            for bb in range(8):
                for t in range(16):
                    r = bb * 16 + t
                    vals = plsc.load_gather(
                        tb.at[s], [lanes, jnp.full((16,), r, jnp.int32)])
                    vb[so, r >> 3, pl.ds((r % 8) * 16, 16)] = vals
            _, dst = _in_slices(p)
            pltpu.async_copy(vb.at[so], dst, osem)
        return c

    lax.fori_loop(0, NTI, _tile, 0)

    for k in (NOB, 2, 1):
        i = NTI - k

        @pl.when((i >= 0) & (wid + i * NW < NTILE))
        def _(i=i):
            p = wid + i * NW
            _, dst = _in_slices(p)
            pltpu.make_async_copy(vb.at[i % NOB], dst, osem).wait()


@jax.jit
def _fmt(tt, catT, xT, tailq):
    mesh = plsc.VectorSubcoreMesh(core_axis_name="c", subcore_axis_name="s")
    return pl.kernel(
        _fmt_body,
        out_type=(
            jax.ShapeDtypeStruct((CAT * RPF, 128), jnp.float32),
            jax.ShapeDtypeStruct((NBT * 32, 128), jnp.int32),
            jax.ShapeDtypeStruct((NBT * 16, 128), jnp.float32),
        ),
        mesh=mesh,
        scratch_types=[
            pltpu.VMEM((NIB, D, 128), jnp.float32),
            pltpu.VMEM((NOB, D, 128), jnp.float32),
            pltpu.VMEM((32, 128), jnp.int32),
            pltpu.VMEM((16, 128), jnp.float32),
            pltpu.VMEM((CAT * 4, 128), jnp.float32),
            pltpu.VMEM((8, 128), jnp.float32),
            pltpu.SemaphoreType.DMA,
            pltpu.SemaphoreType.DMA,
            pltpu.SemaphoreType.DMA,
        ],
        compiler_params=pltpu.CompilerParams(use_tc_tiling_on_sc=True,
                                             needs_layout_passes=False),
    )(tt, catT, xT, tailq)


def _tok_body(xt_hbm, idxt_hbm, tbl_hbm, w_hbm, bias_hbm, out_hbm,
              idxt_v, xt_v, w_v, bias_v, gbuf, stage, gsem0, gsem1, osem):
    wid = lax.axis_index("s") * NC + lax.axis_index("c")
    base = wid * BPW

    pltpu.sync_copy(w_hbm, w_v)
    pltpu.sync_copy(bias_hbm, bias_v)

    lanes = jax.lax.iota(jnp.int32, 16)
    lanes3 = lanes >> 3
    icol = (lanes & 7) * 128
    gsems = (gsem0, gsem1)

    def _fire(h, hb):
        off = pl.multiple_of(h * H, H)
        for j in range(CAT):
            pltpu.async_copy(tbl_hbm.at[idxt_v.at[j, pl.ds(off, H)]],
                             gbuf.at[hb, j], gsems[hb])

    def _drain(h, hb):
        off = pl.multiple_of(h * H, H)
        for j in range(CAT):
            pltpu.make_async_copy(tbl_hbm.at[idxt_v.at[j, pl.ds(off, H)]],
                                  gbuf.at[hb, j], gsems[hb]).wait()

    def _group(g, carry):
        btv = wid * NGRP + g
        pltpu.sync_copy(idxt_hbm.at[btv, pl.ds(0, CAT)], idxt_v)
        pltpu.sync_copy(xt_hbm.at[btv, pl.ds(0, NUM)], xt_v)

        for j in range(CAT):
            for kk in range(G // 16):
                sl = pl.ds(kk * 16, 16)
                idxt_v[j, sl] = idxt_v[j, sl] + j * VPAD

        _fire(0, 0)

        @pl.when(g > 0)
        def _():
            pltpu.make_async_copy(
                stage, out_hbm.at[:, wid * NGRP + g - 1],
                osem).wait()

        for n in range(NUM):
            wv = w_v[n]
            bv = bias_v[n]
            for dt in range(2):
                for din in range(8):
                    d = dt * 8 + din
                    sw = wv[d]
                    sb = bv[d]

                    def _numk(k, c, n=n, dt=dt, din=din, sw=sw, sb=sb):
                        koff = pl.multiple_of(k * 16, 16)
                        xv = xt_v[n, pl.ds(koff, 16)]
                        stage[n * 2 + dt,
                              pl.ds(din * 128 + k * 16, 16)] = xv * sw + sb
                        return c
                    lax.fori_loop(0, G // 16, _numk, 0)

        for h in range(NH):
            hb = h % 2
            _drain(h, hb)
            if h + 1 < NH:
                _fire(h + 1, (h + 1) % 2)

            def _scat(bb, c, h=h, hb=hb):
                colv = icol + (h * H + bb)
                for j in range(CAT):
                    rowv = lanes3 + (NUM + j) * 2
                    plsc.store_scatter(stage, [rowv, colv], gbuf[hb, j, bb])
                return c
            lax.fori_loop(0, H, _scat, 0)

        pltpu.async_copy(stage, out_hbm.at[:, wid * NGRP + g],
                         osem)
        return carry

    lax.fori_loop(0, NGRP, _group, 0)
    pltpu.make_async_copy(
        stage, out_hbm.at[:, wid * NGRP + NGRP - 1], osem).wait()


@jax.jit
def _tok(xt, idxt, tbl, W, b):
    mesh = plsc.VectorSubcoreMesh(core_axis_name="c", subcore_axis_name="s")
    return pl.kernel(
        _tok_body,
        out_type=jax.ShapeDtypeStruct((TOK * 2, B // G, 8 * 128), jnp.float32),
        mesh=mesh,
        scratch_types=[
            pltpu.VMEM((CAT, G), jnp.int32),
            pltpu.VMEM((NUM, G), jnp.float32),
            pltpu.VMEM((NUM, D), jnp.float32),
            pltpu.VMEM((NUM, D), jnp.float32),
            pltpu.VMEM((2, CAT, H, D), jnp.float32),
            pltpu.VMEM((TOK * 2, 8 * 128), jnp.float32),
            pltpu.SemaphoreType.DMA,
            pltpu.SemaphoreType.DMA,
            pltpu.SemaphoreType.DMA,
        ],
        compiler_params=pltpu.CompilerParams(use_tc_tiling_on_sc=False,
                                             needs_layout_passes=False),
    )(xt, idxt, tbl, W, b)


def kernel(numerical_features, categorical_features, tables, W, b):
    tt = tables.transpose(0, 2, 1)
    catT = categorical_features.T
    xT = numerical_features.T
    tailq = lax.slice(tables, (0, V - VTAIL, 0), (CAT, V, D))
    tailq = tailq.reshape(CAT * VTAIL * D // 128, 128)
    tblf, idxt, xt = _fmt(tt, catT, xT, tailq)
    out = _tok(xt.reshape(B // G, 16, 128), idxt.reshape(B // G, 32, 128),
               tblf.reshape(CAT * VPAD, D), W, b)
    out = out.reshape(TOK, 2, B // G, 8, 128)
    out = out.transpose(2, 4, 0, 1, 3)
    return out.reshape(B, TOK, D)


# speedup vs baseline: 2.3759x; 2.0885x over previous
import jax
import jax.numpy as jnp
from jax import lax
from jax.experimental import pallas as pl
from jax.experimental.pallas import tpu as pltpu
from jax.experimental.pallas import tpu_sc as plsc

B = 16384
NUM = 13
CAT = 26
TOK = NUM + CAT
V = 100000
D = 16

_info = plsc.get_sparse_core_info()
NC, NS = _info.num_cores, _info.num_subcores
NW = NC * NS
BPW = B // NW
G = 128
NGRP = BPW // G
H = 32
NH = G // H

NFT = V // 128
VTAIL = V - NFT * 128
RPF = 12504
VPAD = RPF * 128 // D
NTILE = CAT * NFT
NTI = (NTILE + NW - 1) // NW
NBT = B // 128
NIB = 2
NOB = 2
NBP = NBT // NW


def _fmt_body(cat_hbm, num_hbm, idxt_hbm, xt_hbm, cbuf, nbuf, csem):
    wid = lax.axis_index("s") * NC + lax.axis_index("c")

    def _cmove(i, c):
        bt = wid + i * NW
        pltpu.sync_copy(cat_hbm.at[:, pl.ds(bt * 128, 128)],
                        cbuf.at[pl.ds(0, CAT)])
        pltpu.sync_copy(num_hbm.at[:, pl.ds(bt * 128, 128)],
                        nbuf.at[pl.ds(0, NUM)])
        pltpu.async_copy(cbuf, idxt_hbm.at[pl.ds(bt * 32, 32)], csem)
        pltpu.async_copy(nbuf, xt_hbm.at[pl.ds(bt * 16, 16)], csem)
        pltpu.make_async_copy(cbuf, idxt_hbm.at[pl.ds(bt * 32, 32)],
                              csem).wait()
        pltpu.make_async_copy(nbuf, xt_hbm.at[pl.ds(bt * 16, 16)],
                              csem).wait()
        return c
    lax.fori_loop(0, NBP, _cmove, 0)


@jax.jit
def _fmt(catT, xT):
    mesh = plsc.VectorSubcoreMesh(core_axis_name="c", subcore_axis_name="s")
    return pl.kernel(
        _fmt_body,
        out_type=(
            jax.ShapeDtypeStruct((NBT * 32, 128), jnp.int32),
            jax.ShapeDtypeStruct((NBT * 16, 128), jnp.float32),
        ),
        mesh=mesh,
        scratch_types=[
            pltpu.VMEM((32, 128), jnp.int32),
            pltpu.VMEM((16, 128), jnp.float32),
            pltpu.SemaphoreType.DMA,
        ],
        compiler_params=pltpu.CompilerParams(use_tc_tiling_on_sc=True,
                                             needs_layout_passes=False),
    )(catT, xT)


def _tok_body(xt_hbm, idxt_hbm, tbl_hbm, w_hbm, bias_hbm, out_hbm,
              idxt_v, xt_v, w_v, bias_v, gbuf, stage, gsem0, gsem1, osem):
    wid = lax.axis_index("s") * NC + lax.axis_index("c")
    base = wid * BPW

    pltpu.sync_copy(w_hbm, w_v)
    pltpu.sync_copy(bias_hbm, bias_v)

    lanes = jax.lax.iota(jnp.int32, 16)
    lanes3 = lanes >> 3
    icol = (lanes & 7) * 128
    gsems = (gsem0, gsem1)

    def _fire(h, hb):
        off = pl.multiple_of(h * H, H)

        def _fj(j, c):
            for d in range(D):
                pltpu.async_copy(
                    tbl_hbm.at[j, d].at[idxt_v.at[j, pl.ds(off, H)]],
                    gbuf.at[hb, j, d], gsems[hb])
            return c
        lax.fori_loop(0, CAT, _fj, 0)

    def _drain(h, hb):
        off = pl.multiple_of(h * H, H)

        def _dj(j, c):
            for d in range(D):
                pltpu.make_async_copy(
                    tbl_hbm.at[j, d].at[idxt_v.at[j, pl.ds(off, H)]],
                    gbuf.at[hb, j, d], gsems[hb]).wait()
            return c
        lax.fori_loop(0, CAT, _dj, 0)

    def _group(g, carry):
        btv = wid * NGRP + g
        pltpu.sync_copy(idxt_hbm.at[btv, pl.ds(0, CAT)], idxt_v)
        pltpu.sync_copy(xt_hbm.at[btv, pl.ds(0, NUM)], xt_v)

        _fire(0, 0)

        @pl.when(g > 0)
        def _():
            pltpu.make_async_copy(
                stage, out_hbm.at[:, wid * NGRP + g - 1],
                osem).wait()

        for n in range(NUM):
            wv = w_v[n]
            bv = bias_v[n]
            for dt in range(2):
                for din in range(8):
                    d = dt * 8 + din
                    sw = wv[d]
                    sb = bv[d]

                    def _numk(k, c, n=n, dt=dt, din=din, sw=sw, sb=sb):
                        koff = pl.multiple_of(k * 16, 16)
                        xv = xt_v[n, pl.ds(koff, 16)]
                        stage[n * 2 + dt,
                              pl.ds(din * 128 + k * 16, 16)] = xv * sw + sb
                        return c
                    lax.fori_loop(0, G // 16, _numk, 0)

        for h in range(NH):
            hb = h % 2
            _drain(h, hb)
            if h + 1 < NH:
                _fire(h + 1, (h + 1) % 2)

            def _asm(j, c, h=h, hb=hb):
                for dt in range(2):
                    for din in range(8):
                        d = dt * 8 + din
                        for kk in range(H // 16):
                            stage[(NUM + j) * 2 + dt,
                                  pl.ds(din * 128 + h * H + kk * 16, 16)] = (
                                      gbuf[hb, j, d, pl.ds(kk * 16, 16)])
                return c
            lax.fori_loop(0, CAT, _asm, 0)

        pltpu.async_copy(stage, out_hbm.at[:, wid * NGRP + g],
                         osem)
        return carry

    lax.fori_loop(0, NGRP, _group, 0)
    pltpu.make_async_copy(
        stage, out_hbm.at[:, wid * NGRP + NGRP - 1], osem).wait()


@jax.jit
def _tok(xt, idxt, tbl, W, b):
    mesh = plsc.VectorSubcoreMesh(core_axis_name="c", subcore_axis_name="s")
    return pl.kernel(
        _tok_body,
        out_type=jax.ShapeDtypeStruct((TOK * 2, B // G, 8 * 128), jnp.float32),
        mesh=mesh,
        scratch_types=[
            pltpu.VMEM((CAT, G), jnp.int32),
            pltpu.VMEM((NUM, G), jnp.float32),
            pltpu.VMEM((NUM, D), jnp.float32),
            pltpu.VMEM((NUM, D), jnp.float32),
            pltpu.VMEM((2, CAT, D, H), jnp.float32),
            pltpu.VMEM((TOK * 2, 8 * 128), jnp.float32),
            pltpu.SemaphoreType.DMA,
            pltpu.SemaphoreType.DMA,
            pltpu.SemaphoreType.DMA,
        ],
        compiler_params=pltpu.CompilerParams(use_tc_tiling_on_sc=False,
                                             needs_layout_passes=False),
    )(xt, idxt, tbl, W, b)


def kernel(numerical_features, categorical_features, tables, W, b):
    tt = tables.transpose(0, 2, 1)
    catT = categorical_features.T
    xT = numerical_features.T
    idxt, xt = _fmt(catT, xT)
    out = _tok(xt.reshape(B // G, 16, 128), idxt.reshape(B // G, 32, 128),
               tt, W, b)
    out = out.reshape(TOK, 2, B // G, 8, 128)
    out = out.transpose(2, 4, 0, 1, 3)
    return out.reshape(B, TOK, D)
